# baseline (speedup 1.0000x reference)
import functools

import jax
import jax.numpy as jnp
from jax import lax
from jax.experimental import pallas as pl
from jax.experimental.pallas import tpu as pltpu
from jax.experimental.pallas import tpu_sc as plsc

D = 128
N_NODE = 10000
PAD_N = 10240
NS = 16
RPT = PAD_N // NS
CH = 96
NB = 8
E = 320000
T_CH = 216
EPT = T_CH * CH
E_PAD = EPT * NS
NBLK = T_CH // NB
CPT = E_PAD // CH
FS = 16

_mesh = plsc.VectorSubcoreMesh(core_axis_name="c", subcore_axis_name="s")


@functools.partial(
    pl.kernel,
    out_type=[jax.ShapeDtypeStruct((2 * PAD_N, D), jnp.float32),
              jax.ShapeDtypeStruct((2, NS, PAD_N), jnp.float32)],
    mesh=_mesh,
    compiler_params=pltpu.CompilerParams(needs_layout_passes=False),
    scratch_types=[
        pltpu.VMEM((NB * CH,), jnp.int32),
        pltpu.VMEM((NB, CH), jnp.int32),
        pltpu.VMEM((CH, D), jnp.float32),
        pltpu.VMEM((FS, D), jnp.float32),
        pltpu.VMEM((PAD_N,), jnp.float32),
        pltpu.VMEM_SHARED((PAD_N, D), jnp.float32),
        pltpu.SemaphoreType.DMA,
    ],
)
def _sc_aggregate(src_all, dst2d_all, x_all, zeros_f, zeros_n,
                  mean_all, cnt_pub,
                  idx_s, idx_d, rows, obuf, cnt_loc, acc_sh, sem):
    c = lax.axis_index("c")
    s = lax.axis_index("s")
    rbase = pl.multiple_of(s * RPT, 8)
    obase = pl.multiple_of(c * PAD_N + s * RPT, 8)
    ebase = pl.multiple_of(c * E_PAD + s * EPT, 8)
    cbase = c * CPT + s * T_CH

    pltpu.sync_copy(zeros_f, obuf)

    def zero_slab(q, carry):
        rb = pl.multiple_of(rbase + q * FS, 8)
        pltpu.sync_copy(obuf, acc_sh.at[pl.ds(rb, FS)])
        return carry

    lax.fori_loop(0, RPT // FS, zero_slab, 0)
    pltpu.sync_copy(zeros_n, cnt_loc)
    plsc.subcore_barrier()

    ones = jnp.full((16,), 1.0, jnp.float32)

    def blk(b, carry):
        eb = pl.multiple_of(ebase + b * (NB * CH), 8)
        pltpu.sync_copy(src_all.at[pl.ds(eb, NB * CH)], idx_s)
        pltpu.sync_copy(dst2d_all.at[pl.ds(cbase + b * NB, NB)], idx_d)
        for jj in range(NB):
            pltpu.async_copy(x_all.at[idx_s.at[pl.ds(jj * CH, CH)]],
                             rows, sem).wait()
            pltpu.sync_copy(rows, acc_sh.at[idx_d.at[jj]], add=True)
            for k in range(CH // 16):
                v = idx_d[jj, pl.ds(k * 16, 16)]
                plsc.addupdate_scatter(cnt_loc, [v], ones)
        return carry

    lax.fori_loop(0, NBLK, blk, 0)
    pltpu.sync_copy(cnt_loc, cnt_pub.at[c, s])
    plsc.subcore_barrier()

    pltpu.sync_copy(cnt_pub.at[c, 0, pl.ds(rbase, RPT)],
                    cnt_loc.at[pl.ds(0, RPT)])
    for t in range(1, NS):
        pltpu.sync_copy(cnt_pub.at[c, t, pl.ds(rbase, RPT)],
                        cnt_loc.at[pl.ds(RPT, RPT)])
        for m in range(RPT // 16):
            cnt_loc[pl.ds(m * 16, 16)] = (
                cnt_loc[pl.ds(m * 16, 16)]
                + cnt_loc[pl.ds(RPT + m * 16, 16)])
    for m in range(RPT // 16):
        cnt_loc[pl.ds(m * 16, 16)] = 1.0 / jnp.maximum(
            cnt_loc[pl.ds(m * 16, 16)], 1.0)

    def fin_slab(q, carry):
        rb = pl.multiple_of(rbase + q * FS, 8)
        pltpu.sync_copy(acc_sh.at[pl.ds(rb, FS)], obuf)
        for j in range(FS):
            ridx = jnp.full((16,), q * FS + j, jnp.int32)
            inv = plsc.load_gather(cnt_loc, [ridx])
            for k in range(D // 16):
                obuf[j, pl.ds(k * 16, 16)] = obuf[j, pl.ds(k * 16, 16)] * inv
        ob = pl.multiple_of(obase + q * FS, 8)
        pltpu.sync_copy(obuf, mean_all.at[pl.ds(ob, FS)])
        return carry

    lax.fori_loop(0, RPT // FS, fin_slab, 0)


def _dense_body(mean_ref, x_ref, wr_ref, br_ref, wt_ref, o_ref):
    dn = (((1,), (1,)), ((), ()))
    o_ref[...] = (
        lax.dot_general(mean_ref[...], wr_ref[...], dn,
                        preferred_element_type=jnp.float32)
        + br_ref[...]
        + lax.dot_general(x_ref[...], wt_ref[...], dn,
                          preferred_element_type=jnp.float32)
    )


def _dense(mean, x_dst, W_rel, b_rel, W_root):
    blk = 1000
    return pl.pallas_call(
        _dense_body,
        grid=(N_NODE // blk,),
        in_specs=[
            pl.BlockSpec((blk, D), lambda i: (i, 0)),
            pl.BlockSpec((blk, D), lambda i: (i, 0)),
            pl.BlockSpec((D, D), lambda i: (0, 0)),
            pl.BlockSpec((1, D), lambda i: (0, 0)),
            pl.BlockSpec((D, D), lambda i: (0, 0)),
        ],
        out_specs=pl.BlockSpec((blk, D), lambda i: (i, 0)),
        out_shape=jax.ShapeDtypeStruct((N_NODE, D), jnp.float32),
    )(mean, x_dst, W_rel, b_rel.reshape(1, D), W_root)


def kernel(x_user, x_item, edge_index_user_item, edge_index_item_user,
           W_rel_ui, b_rel_ui, W_root_ui, W_rel_iu, b_rel_iu, W_root_iu):
    pad = E_PAD - E
    pad_dst = jnp.full((pad,), PAD_N - 1, jnp.int32)

    def pad_type(edge_index, src_off):
        src = jnp.concatenate([edge_index[0].astype(jnp.int32) + src_off,
                               jnp.full((pad,), src_off, jnp.int32)])
        dst = jnp.concatenate([edge_index[1].astype(jnp.int32), pad_dst])
        return src, dst

    src_ui, dst_ui = pad_type(edge_index_user_item, 0)
    src_iu, dst_iu = pad_type(edge_index_item_user, N_NODE)
    src_all = jnp.concatenate([src_ui, src_iu])
    dst2d_all = jnp.concatenate([dst_ui, dst_iu]).reshape(2 * CPT, CH)
    x_all = jnp.concatenate([x_user, x_item])

    zeros_f = jnp.zeros((FS, D), jnp.float32)
    zeros_n = jnp.zeros((PAD_N,), jnp.float32)

    mean_all, _ = _sc_aggregate(src_all, dst2d_all, x_all, zeros_f, zeros_n)
    mean_ui = mean_all[:N_NODE]
    mean_iu = mean_all[PAD_N:PAD_N + N_NODE]

    out_item = _dense(mean_ui, x_item, W_rel_ui, b_rel_ui, W_root_ui)
    out_user = _dense(mean_iu, x_user, W_rel_iu, b_rel_iu, W_root_iu)
    return (out_user, out_item)


# speedup vs baseline: 2.1115x; 2.1115x over previous
import functools

import jax
import jax.numpy as jnp
from jax import lax
from jax.experimental import pallas as pl
from jax.experimental.pallas import tpu as pltpu
from jax.experimental.pallas import tpu_sc as plsc

D = 128
N_NODE = 10000
PAD_N = 10240
NS = 16
RPT = PAD_N // NS
CH = 64
E = 320000
T_CH = 314
EPT = T_CH * CH
E_PAD = EPT * NS
CPT = E_PAD // CH
FS = 16

_mesh = plsc.VectorSubcoreMesh(core_axis_name="c", subcore_axis_name="s")


@functools.partial(
    pl.kernel,
    out_type=[jax.ShapeDtypeStruct((2 * PAD_N, D), jnp.float32),
              jax.ShapeDtypeStruct((2, NS, PAD_N), jnp.float32)],
    mesh=_mesh,
    compiler_params=pltpu.CompilerParams(needs_layout_passes=False),
    scratch_types=[
        pltpu.VMEM((CH,), jnp.int32),
        pltpu.VMEM((CH,), jnp.int32),
        pltpu.VMEM((CH,), jnp.int32),
        pltpu.VMEM((CH,), jnp.int32),
        pltpu.VMEM((CH, D), jnp.float32),
        pltpu.VMEM((CH, D), jnp.float32),
        pltpu.VMEM((FS, D), jnp.float32),
        pltpu.VMEM((PAD_N,), jnp.float32),
        pltpu.VMEM_SHARED((PAD_N, D), jnp.float32),
        pltpu.SemaphoreType.DMA,
        pltpu.SemaphoreType.DMA,
    ],
)
def _sc_aggregate(src_all, dst_all, x_all, zeros_f, zeros_n,
                  mean_all, cnt_pub,
                  idx_sa, idx_da, idx_sb, idx_db, rows_a, rows_b,
                  obuf, cnt_loc, acc_sh, sem_a, sem_b):
    c = lax.axis_index("c")
    s = lax.axis_index("s")
    rbase = pl.multiple_of(s * RPT, 8)
    obase = pl.multiple_of(c * PAD_N + s * RPT, 8)
    ebase = pl.multiple_of(c * E_PAD + s * EPT, 8)

    pltpu.sync_copy(zeros_f, obuf)

    def zero_slab(q, carry):
        rb = pl.multiple_of(rbase + q * FS, 8)
        pltpu.sync_copy(obuf, acc_sh.at[pl.ds(rb, FS)])
        return carry

    lax.fori_loop(0, RPT // FS, zero_slab, 0)
    pltpu.sync_copy(zeros_n, cnt_loc)
    plsc.subcore_barrier()

    ones = jnp.full((16,), 1.0, jnp.float32)

    def prefetch(i, idx_s, idx_d, rows, sem):
        base = pl.multiple_of(ebase + i * CH, 8)
        pltpu.sync_copy(src_all.at[pl.ds(base, CH)], idx_s)
        pltpu.async_copy(x_all.at[idx_s], rows, sem)
        pltpu.sync_copy(dst_all.at[pl.ds(base, CH)], idx_d)

    def finish(idx_d, rows, sem):
        pltpu.make_async_copy(x_all.at[pl.ds(0, CH)], rows, sem).wait()
        pltpu.sync_copy(rows, acc_sh.at[idx_d], add=True)
        for k in range(CH // 16):
            v = idx_d[pl.ds(k * 16, 16)]
            plsc.addupdate_scatter(cnt_loc, [v], ones)

    prefetch(0, idx_sa, idx_da, rows_a, sem_a)

    def pair(q, carry):
        prefetch(2 * q + 1, idx_sb, idx_db, rows_b, sem_b)
        finish(idx_da, rows_a, sem_a)
        nxt = jnp.minimum(2 * q + 2, T_CH - 1)
        prefetch(nxt, idx_sa, idx_da, rows_a, sem_a)
        finish(idx_db, rows_b, sem_b)
        return carry

    lax.fori_loop(0, T_CH // 2, pair, 0)
    pltpu.make_async_copy(x_all.at[pl.ds(0, CH)], rows_a, sem_a).wait()
    pltpu.sync_copy(cnt_loc, cnt_pub.at[c, s])
    plsc.subcore_barrier()

    pltpu.sync_copy(cnt_pub.at[c, 0, pl.ds(rbase, RPT)],
                    cnt_loc.at[pl.ds(0, RPT)])
    for t in range(1, NS):
        pltpu.sync_copy(cnt_pub.at[c, t, pl.ds(rbase, RPT)],
                        cnt_loc.at[pl.ds(RPT, RPT)])
        for m in range(RPT // 16):
            cnt_loc[pl.ds(m * 16, 16)] = (
                cnt_loc[pl.ds(m * 16, 16)]
                + cnt_loc[pl.ds(RPT + m * 16, 16)])
    for m in range(RPT // 16):
        cnt_loc[pl.ds(m * 16, 16)] = 1.0 / jnp.maximum(
            cnt_loc[pl.ds(m * 16, 16)], 1.0)

    def fin_slab(q, carry):
        rb = pl.multiple_of(rbase + q * FS, 8)
        pltpu.sync_copy(acc_sh.at[pl.ds(rb, FS)], obuf)
        for j in range(FS):
            ridx = jnp.full((16,), q * FS + j, jnp.int32)
            inv = plsc.load_gather(cnt_loc, [ridx])
            for k in range(D // 16):
                obuf[j, pl.ds(k * 16, 16)] = obuf[j, pl.ds(k * 16, 16)] * inv
        ob = pl.multiple_of(obase + q * FS, 8)
        pltpu.sync_copy(obuf, mean_all.at[pl.ds(ob, FS)])
        return carry

    lax.fori_loop(0, RPT // FS, fin_slab, 0)


def _dense_body(mean_ref, x_ref, wr_ref, br_ref, wt_ref, o_ref):
    dn = (((1,), (1,)), ((), ()))
    o_ref[...] = (
        lax.dot_general(mean_ref[...], wr_ref[...], dn,
                        preferred_element_type=jnp.float32)
        + br_ref[...]
        + lax.dot_general(x_ref[...], wt_ref[...], dn,
                          preferred_element_type=jnp.float32)
    )


def _dense(mean, x_dst, W_rel, b_rel, W_root):
    blk = 1000
    return pl.pallas_call(
        _dense_body,
        grid=(N_NODE // blk,),
        in_specs=[
            pl.BlockSpec((blk, D), lambda i: (i, 0)),
            pl.BlockSpec((blk, D), lambda i: (i, 0)),
            pl.BlockSpec((D, D), lambda i: (0, 0)),
            pl.BlockSpec((1, D), lambda i: (0, 0)),
            pl.BlockSpec((D, D), lambda i: (0, 0)),
        ],
        out_specs=pl.BlockSpec((blk, D), lambda i: (i, 0)),
        out_shape=jax.ShapeDtypeStruct((N_NODE, D), jnp.float32),
    )(mean, x_dst, W_rel, b_rel.reshape(1, D), W_root)


def kernel(x_user, x_item, edge_index_user_item, edge_index_item_user,
           W_rel_ui, b_rel_ui, W_root_ui, W_rel_iu, b_rel_iu, W_root_iu):
    pad = E_PAD - E
    pad_dst = jnp.full((pad,), PAD_N - 1, jnp.int32)

    def pad_type(edge_index, src_off):
        src = jnp.concatenate([edge_index[0].astype(jnp.int32) + src_off,
                               jnp.full((pad,), src_off, jnp.int32)])
        dst = jnp.concatenate([edge_index[1].astype(jnp.int32), pad_dst])
        return src, dst

    src_ui, dst_ui = pad_type(edge_index_user_item, 0)
    src_iu, dst_iu = pad_type(edge_index_item_user, N_NODE)
    src_all = jnp.concatenate([src_ui, src_iu])
    dst_all = jnp.concatenate([dst_ui, dst_iu])
    x_all = jnp.concatenate([x_user, x_item])

    zeros_f = jnp.zeros((FS, D), jnp.float32)
    zeros_n = jnp.zeros((PAD_N,), jnp.float32)

    mean_all, _ = _sc_aggregate(src_all, dst_all, x_all, zeros_f, zeros_n)
    mean_ui = mean_all[:N_NODE]
    mean_iu = mean_all[PAD_N:PAD_N + N_NODE]

    out_item = _dense(mean_ui, x_item, W_rel_ui, b_rel_ui, W_root_ui)
    out_user = _dense(mean_iu, x_user, W_rel_iu, b_rel_iu, W_root_iu)
    return (out_user, out_item)


# speedup vs baseline: 2.5217x; 1.1943x over previous
import functools

import jax
import jax.numpy as jnp
from jax import lax
from jax.experimental import pallas as pl
from jax.experimental.pallas import tpu as pltpu
from jax.experimental.pallas import tpu_sc as plsc

D = 128
N_NODE = 10000
PAD_N = 10240
NS = 16
RPT = PAD_N // NS
CH = 80
E = 320000
T_CH = 250
EPT = T_CH * CH
E_PAD = EPT * NS
CPT = E_PAD // CH
FS = 16

_mesh = plsc.VectorSubcoreMesh(core_axis_name="c", subcore_axis_name="s")


@functools.partial(
    pl.kernel,
    out_type=[jax.ShapeDtypeStruct((2 * PAD_N, D), jnp.float32),
              jax.ShapeDtypeStruct((2, NS, PAD_N), jnp.float32)],
    mesh=_mesh,
    compiler_params=pltpu.CompilerParams(needs_layout_passes=False),
    scratch_types=[
        pltpu.VMEM((CH,), jnp.int32),
        pltpu.VMEM((CH,), jnp.int32),
        pltpu.VMEM((CH,), jnp.int32),
        pltpu.VMEM((CH,), jnp.int32),
        pltpu.VMEM((CH, D), jnp.float32),
        pltpu.VMEM((CH, D), jnp.float32),
        pltpu.VMEM((PAD_N,), jnp.float32),
        pltpu.VMEM_SHARED((PAD_N, D), jnp.float32),
        pltpu.SemaphoreType.DMA,
        pltpu.SemaphoreType.DMA,
    ],
)
def _sc_aggregate(src_all, dst_all, x_all, zeros_f, zeros_n,
                  mean_all, cnt_pub,
                  idx_sa, idx_da, idx_sb, idx_db, rows_a, rows_b,
                  cnt_loc, acc_sh, sem_a, sem_b):
    obuf = rows_a.at[pl.ds(0, FS)]
    c = lax.axis_index("c")
    s = lax.axis_index("s")
    rbase = pl.multiple_of(s * RPT, 8)
    obase = pl.multiple_of(c * PAD_N + s * RPT, 8)
    ebase = pl.multiple_of(c * E_PAD + s * EPT, 8)

    pltpu.sync_copy(zeros_f, obuf)

    def zero_slab(q, carry):
        rb = pl.multiple_of(rbase + q * FS, 8)
        pltpu.sync_copy(obuf, acc_sh.at[pl.ds(rb, FS)])
        return carry

    lax.fori_loop(0, RPT // FS, zero_slab, 0)
    pltpu.sync_copy(zeros_n, cnt_loc)
    plsc.subcore_barrier()

    ones = jnp.full((16,), 1.0, jnp.float32)

    def prefetch(i, idx_s, idx_d, rows, sem):
        base = pl.multiple_of(ebase + i * CH, 8)
        pltpu.sync_copy(src_all.at[pl.ds(base, CH)], idx_s)
        pltpu.async_copy(x_all.at[idx_s], rows, sem)
        pltpu.sync_copy(dst_all.at[pl.ds(base, CH)], idx_d)

    def finish(idx_d, rows, sem):
        pltpu.make_async_copy(x_all.at[pl.ds(0, CH)], rows, sem).wait()
        pltpu.sync_copy(rows, acc_sh.at[idx_d], add=True)
        for k in range(CH // 16):
            v = idx_d[pl.ds(k * 16, 16)]
            plsc.addupdate_scatter(cnt_loc, [v], ones)

    prefetch(0, idx_sa, idx_da, rows_a, sem_a)

    def pair(q, carry):
        prefetch(2 * q + 1, idx_sb, idx_db, rows_b, sem_b)
        finish(idx_da, rows_a, sem_a)
        nxt = jnp.minimum(2 * q + 2, T_CH - 1)
        prefetch(nxt, idx_sa, idx_da, rows_a, sem_a)
        finish(idx_db, rows_b, sem_b)
        return carry

    lax.fori_loop(0, T_CH // 2, pair, 0)
    pltpu.make_async_copy(x_all.at[pl.ds(0, CH)], rows_a, sem_a).wait()
    pltpu.sync_copy(cnt_loc, cnt_pub.at[c, s])
    plsc.subcore_barrier()

    pltpu.sync_copy(cnt_pub.at[c, 0, pl.ds(rbase, RPT)],
                    cnt_loc.at[pl.ds(0, RPT)])
    for t in range(1, NS):
        pltpu.sync_copy(cnt_pub.at[c, t, pl.ds(rbase, RPT)],
                        cnt_loc.at[pl.ds(RPT, RPT)])
        for m in range(RPT // 16):
            cnt_loc[pl.ds(m * 16, 16)] = (
                cnt_loc[pl.ds(m * 16, 16)]
                + cnt_loc[pl.ds(RPT + m * 16, 16)])
    for m in range(RPT // 16):
        cnt_loc[pl.ds(m * 16, 16)] = 1.0 / jnp.maximum(
            cnt_loc[pl.ds(m * 16, 16)], 1.0)

    def fin_slab(q, carry):
        rb = pl.multiple_of(rbase + q * FS, 8)
        pltpu.sync_copy(acc_sh.at[pl.ds(rb, FS)], obuf)
        for j in range(FS):
            ridx = jnp.full((16,), q * FS + j, jnp.int32)
            inv = plsc.load_gather(cnt_loc, [ridx])
            for k in range(D // 16):
                obuf[j, pl.ds(k * 16, 16)] = obuf[j, pl.ds(k * 16, 16)] * inv
        ob = pl.multiple_of(obase + q * FS, 8)
        pltpu.sync_copy(obuf, mean_all.at[pl.ds(ob, FS)])
        return carry

    lax.fori_loop(0, RPT // FS, fin_slab, 0)


def _dense_body(mean_ref, x_ref, wr_ref, br_ref, wt_ref, o_ref):
    dn = (((1,), (1,)), ((), ()))
    o_ref[...] = (
        lax.dot_general(mean_ref[...], wr_ref[...], dn,
                        preferred_element_type=jnp.float32)
        + br_ref[...]
        + lax.dot_general(x_ref[...], wt_ref[...], dn,
                          preferred_element_type=jnp.float32)
    )


def _dense(mean, x_dst, W_rel, b_rel, W_root):
    blk = 1000
    return pl.pallas_call(
        _dense_body,
        grid=(N_NODE // blk,),
        in_specs=[
            pl.BlockSpec((blk, D), lambda i: (i, 0)),
            pl.BlockSpec((blk, D), lambda i: (i, 0)),
            pl.BlockSpec((D, D), lambda i: (0, 0)),
            pl.BlockSpec((1, D), lambda i: (0, 0)),
            pl.BlockSpec((D, D), lambda i: (0, 0)),
        ],
        out_specs=pl.BlockSpec((blk, D), lambda i: (i, 0)),
        out_shape=jax.ShapeDtypeStruct((N_NODE, D), jnp.float32),
    )(mean, x_dst, W_rel, b_rel.reshape(1, D), W_root)


def kernel(x_user, x_item, edge_index_user_item, edge_index_item_user,
           W_rel_ui, b_rel_ui, W_root_ui, W_rel_iu, b_rel_iu, W_root_iu):
    pad = E_PAD - E
    pad_dst = jnp.full((pad,), PAD_N - 1, jnp.int32)

    def pad_type(edge_index, src_off):
        src = jnp.concatenate([edge_index[0].astype(jnp.int32) + src_off,
                               jnp.full((pad,), src_off, jnp.int32)])
        dst = jnp.concatenate([edge_index[1].astype(jnp.int32), pad_dst])
        return src, dst

    src_ui, dst_ui = pad_type(edge_index_user_item, 0)
    src_iu, dst_iu = pad_type(edge_index_item_user, N_NODE)
    src_all = jnp.concatenate([src_ui, src_iu])
    dst_all = jnp.concatenate([dst_ui, dst_iu])
    x_all = jnp.concatenate([x_user, x_item])

    zeros_f = jnp.zeros((FS, D), jnp.float32)
    zeros_n = jnp.zeros((PAD_N,), jnp.float32)

    mean_all, _ = _sc_aggregate(src_all, dst_all, x_all, zeros_f, zeros_n)
    mean_ui = mean_all[:N_NODE]
    mean_iu = mean_all[PAD_N:PAD_N + N_NODE]

    out_item = _dense(mean_ui, x_item, W_rel_ui, b_rel_ui, W_root_ui)
    out_user = _dense(mean_iu, x_user, W_rel_iu, b_rel_iu, W_root_iu)
    return (out_user, out_item)


# speedup vs baseline: 2.5233x; 1.0007x over previous
import functools

import jax
import jax.numpy as jnp
from jax import lax
from jax.experimental import pallas as pl
from jax.experimental.pallas import tpu as pltpu
from jax.experimental.pallas import tpu_sc as plsc

D = 128
N_NODE = 10000
PAD_N = 10240
NS = 16
RPT = PAD_N // NS
CH = 80
E = 320000
T_CH = 250
EPT = T_CH * CH
E_PAD = EPT * NS
CPT = E_PAD // CH
FS = 16

_mesh = plsc.VectorSubcoreMesh(core_axis_name="c", subcore_axis_name="s")


@functools.partial(
    pl.kernel,
    out_type=[jax.ShapeDtypeStruct((2 * PAD_N, D), jnp.float32),
              jax.ShapeDtypeStruct((2, NS, PAD_N), jnp.float32)],
    mesh=_mesh,
    compiler_params=pltpu.CompilerParams(needs_layout_passes=False),
    scratch_types=[
        pltpu.VMEM((CH,), jnp.int32),
        pltpu.VMEM((CH,), jnp.int32),
        pltpu.VMEM((CH,), jnp.int32),
        pltpu.VMEM((CH,), jnp.int32),
        pltpu.VMEM((CH, D), jnp.float32),
        pltpu.VMEM((CH, D), jnp.float32),
        pltpu.VMEM((PAD_N,), jnp.float32),
        pltpu.VMEM_SHARED((PAD_N, D), jnp.float32),
        pltpu.SemaphoreType.DMA,
        pltpu.SemaphoreType.DMA,
    ],
)
def _sc_aggregate(src_all, dst_all, x_all, zeros_f, zeros_n,
                  mean_all, cnt_pub,
                  idx_sa, idx_da, idx_sb, idx_db, rows_a, rows_b,
                  cnt_loc, acc_sh, sem_a, sem_b):
    obuf = rows_a.at[pl.ds(0, FS)]
    c = lax.axis_index("c")
    s = lax.axis_index("s")
    rbase = pl.multiple_of(s * RPT, 8)
    obase = pl.multiple_of(c * PAD_N + s * RPT, 8)
    ebase = pl.multiple_of(c * E_PAD + s * EPT, 8)

    pltpu.sync_copy(zeros_f, obuf)

    def zero_slab(q, carry):
        rb = pl.multiple_of(rbase + q * FS, 8)
        pltpu.sync_copy(obuf, acc_sh.at[pl.ds(rb, FS)])
        return carry

    lax.fori_loop(0, RPT // FS, zero_slab, 0)
    pltpu.sync_copy(zeros_n, cnt_loc)
    plsc.subcore_barrier()

    ones = jnp.full((16,), 1.0, jnp.float32)

    def prefetch(i, idx_s, idx_d, rows, sem):
        base = pl.multiple_of(ebase + i * CH, 8)
        pltpu.sync_copy(src_all.at[pl.ds(base, CH)], idx_s)
        pltpu.async_copy(x_all.at[idx_s], rows, sem)
        pltpu.sync_copy(dst_all.at[pl.ds(base, CH)], idx_d)

    def finish(idx_d, rows, sem):
        for k in range(CH // 16):
            v = idx_d[pl.ds(k * 16, 16)]
            plsc.addupdate_scatter(cnt_loc, [v], ones)
        pltpu.make_async_copy(x_all.at[pl.ds(0, CH)], rows, sem).wait()
        pltpu.sync_copy(rows, acc_sh.at[idx_d], add=True)

    prefetch(0, idx_sa, idx_da, rows_a, sem_a)

    def pair(q, carry):
        prefetch(2 * q + 1, idx_sb, idx_db, rows_b, sem_b)
        finish(idx_da, rows_a, sem_a)
        nxt = jnp.minimum(2 * q + 2, T_CH - 1)
        prefetch(nxt, idx_sa, idx_da, rows_a, sem_a)
        finish(idx_db, rows_b, sem_b)
        return carry

    lax.fori_loop(0, T_CH // 2, pair, 0)
    pltpu.make_async_copy(x_all.at[pl.ds(0, CH)], rows_a, sem_a).wait()
    pltpu.sync_copy(cnt_loc, cnt_pub.at[c, s])
    plsc.subcore_barrier()

    pltpu.sync_copy(cnt_pub.at[c, 0, pl.ds(rbase, RPT)],
                    cnt_loc.at[pl.ds(0, RPT)])
    for t in range(1, NS):
        pltpu.sync_copy(cnt_pub.at[c, t, pl.ds(rbase, RPT)],
                        cnt_loc.at[pl.ds(RPT, RPT)])
        for m in range(RPT // 16):
            cnt_loc[pl.ds(m * 16, 16)] = (
                cnt_loc[pl.ds(m * 16, 16)]
                + cnt_loc[pl.ds(RPT + m * 16, 16)])
    for m in range(RPT // 16):
        cnt_loc[pl.ds(m * 16, 16)] = 1.0 / jnp.maximum(
            cnt_loc[pl.ds(m * 16, 16)], 1.0)

    def fin_slab(q, carry):
        rb = pl.multiple_of(rbase + q * FS, 8)
        pltpu.sync_copy(acc_sh.at[pl.ds(rb, FS)], obuf)
        for j in range(FS):
            ridx = jnp.full((16,), q * FS + j, jnp.int32)
            inv = plsc.load_gather(cnt_loc, [ridx])
            for k in range(D // 16):
                obuf[j, pl.ds(k * 16, 16)] = obuf[j, pl.ds(k * 16, 16)] * inv
        ob = pl.multiple_of(obase + q * FS, 8)
        pltpu.sync_copy(obuf, mean_all.at[pl.ds(ob, FS)])
        return carry

    lax.fori_loop(0, RPT // FS, fin_slab, 0)


def _dense_body(mean_ref, x_ref, wr_ref, br_ref, wt_ref, o_ref):
    dn = (((1,), (1,)), ((), ()))
    o_ref[...] = (
        lax.dot_general(mean_ref[...], wr_ref[...], dn,
                        preferred_element_type=jnp.float32)
        + br_ref[...]
        + lax.dot_general(x_ref[...], wt_ref[...], dn,
                          preferred_element_type=jnp.float32)
    )


def _dense(mean, x_dst, W_rel, b_rel, W_root):
    blk = 1000
    return pl.pallas_call(
        _dense_body,
        grid=(N_NODE // blk,),
        in_specs=[
            pl.BlockSpec((blk, D), lambda i: (i, 0)),
            pl.BlockSpec((blk, D), lambda i: (i, 0)),
            pl.BlockSpec((D, D), lambda i: (0, 0)),
            pl.BlockSpec((1, D), lambda i: (0, 0)),
            pl.BlockSpec((D, D), lambda i: (0, 0)),
        ],
        out_specs=pl.BlockSpec((blk, D), lambda i: (i, 0)),
        out_shape=jax.ShapeDtypeStruct((N_NODE, D), jnp.float32),
    )(mean, x_dst, W_rel, b_rel.reshape(1, D), W_root)


def kernel(x_user, x_item, edge_index_user_item, edge_index_item_user,
           W_rel_ui, b_rel_ui, W_root_ui, W_rel_iu, b_rel_iu, W_root_iu):
    pad = E_PAD - E
    pad_dst = jnp.full((pad,), PAD_N - 1, jnp.int32)

    def pad_type(edge_index, src_off):
        src = jnp.concatenate([edge_index[0].astype(jnp.int32) + src_off,
                               jnp.full((pad,), src_off, jnp.int32)])
        dst = jnp.concatenate([edge_index[1].astype(jnp.int32), pad_dst])
        return src, dst

    src_ui, dst_ui = pad_type(edge_index_user_item, 0)
    src_iu, dst_iu = pad_type(edge_index_item_user, N_NODE)
    src_all = jnp.concatenate([src_ui, src_iu])
    dst_all = jnp.concatenate([dst_ui, dst_iu])
    x_all = jnp.concatenate([x_user, x_item])

    zeros_f = jnp.zeros((FS, D), jnp.float32)
    zeros_n = jnp.zeros((PAD_N,), jnp.float32)

    mean_all, _ = _sc_aggregate(src_all, dst_all, x_all, zeros_f, zeros_n)
    mean_ui = mean_all[:N_NODE]
    mean_iu = mean_all[PAD_N:PAD_N + N_NODE]

    out_item = _dense(mean_ui, x_item, W_rel_ui, b_rel_ui, W_root_ui)
    out_user = _dense(mean_iu, x_user, W_rel_iu, b_rel_iu, W_root_iu)
    return (out_user, out_item)


# speedup vs baseline: 2.5429x; 1.0078x over previous
import functools

import jax
import jax.numpy as jnp
from jax import lax
from jax.experimental import pallas as pl
from jax.experimental.pallas import tpu as pltpu
from jax.experimental.pallas import tpu_sc as plsc

D = 128
N_NODE = 10000
PAD_N = 10240
NS = 16
RPT = PAD_N // NS
CH = 80
E = 320000
T_CH = 250
EPT = T_CH * CH
E_PAD = EPT * NS
CPT = E_PAD // CH
FS = 80

_mesh = plsc.VectorSubcoreMesh(core_axis_name="c", subcore_axis_name="s")


@functools.partial(
    pl.kernel,
    out_type=[jax.ShapeDtypeStruct((2 * PAD_N, D), jnp.float32),
              jax.ShapeDtypeStruct((2, NS, PAD_N), jnp.float32)],
    mesh=_mesh,
    compiler_params=pltpu.CompilerParams(needs_layout_passes=False),
    scratch_types=[
        pltpu.VMEM((CH,), jnp.int32),
        pltpu.VMEM((CH,), jnp.int32),
        pltpu.VMEM((CH,), jnp.int32),
        pltpu.VMEM((CH,), jnp.int32),
        pltpu.VMEM((CH, D), jnp.float32),
        pltpu.VMEM((CH, D), jnp.float32),
        pltpu.VMEM((PAD_N,), jnp.float32),
        pltpu.VMEM_SHARED((PAD_N, D), jnp.float32),
        pltpu.SemaphoreType.DMA,
        pltpu.SemaphoreType.DMA,
    ],
)
def _sc_aggregate(src_all, dst_all, x_all, zeros_f, zeros_n,
                  mean_all, cnt_pub,
                  idx_sa, idx_da, idx_sb, idx_db, rows_a, rows_b,
                  cnt_loc, acc_sh, sem_a, sem_b):
    obuf = rows_a.at[pl.ds(0, FS)]
    c = lax.axis_index("c")
    s = lax.axis_index("s")
    rbase = pl.multiple_of(s * RPT, 8)
    obase = pl.multiple_of(c * PAD_N + s * RPT, 8)
    ebase = pl.multiple_of(c * E_PAD + s * EPT, 8)

    pltpu.sync_copy(zeros_f, obuf)

    def zero_slab(q, carry):
        rb = pl.multiple_of(rbase + q * FS, 8)
        pltpu.sync_copy(obuf, acc_sh.at[pl.ds(rb, FS)])
        return carry

    lax.fori_loop(0, RPT // FS, zero_slab, 0)
    pltpu.sync_copy(zeros_n, cnt_loc)
    plsc.subcore_barrier()

    ones = jnp.full((16,), 1.0, jnp.float32)

    def prefetch(i, idx_s, idx_d, rows, sem):
        base = pl.multiple_of(ebase + i * CH, 8)
        pltpu.sync_copy(src_all.at[pl.ds(base, CH)], idx_s)
        pltpu.async_copy(x_all.at[idx_s], rows, sem)
        pltpu.sync_copy(dst_all.at[pl.ds(base, CH)], idx_d)

    def finish(idx_d, rows, sem):
        for k in range(CH // 16):
            v = idx_d[pl.ds(k * 16, 16)]
            plsc.addupdate_scatter(cnt_loc, [v], ones)
        pltpu.make_async_copy(x_all.at[pl.ds(0, CH)], rows, sem).wait()
        pltpu.sync_copy(rows, acc_sh.at[idx_d], add=True)

    prefetch(0, idx_sa, idx_da, rows_a, sem_a)

    def pair(q, carry):
        prefetch(2 * q + 1, idx_sb, idx_db, rows_b, sem_b)
        finish(idx_da, rows_a, sem_a)
        nxt = jnp.minimum(2 * q + 2, T_CH - 1)
        prefetch(nxt, idx_sa, idx_da, rows_a, sem_a)
        finish(idx_db, rows_b, sem_b)
        return carry

    lax.fori_loop(0, T_CH // 2, pair, 0)
    pltpu.make_async_copy(x_all.at[pl.ds(0, CH)], rows_a, sem_a).wait()
    pltpu.sync_copy(cnt_loc, cnt_pub.at[c, s])
    plsc.subcore_barrier()

    pltpu.sync_copy(cnt_pub.at[c, 0, pl.ds(rbase, RPT)],
                    cnt_loc.at[pl.ds(0, RPT)])
    for t in range(1, NS):
        pltpu.sync_copy(cnt_pub.at[c, t, pl.ds(rbase, RPT)],
                        cnt_loc.at[pl.ds(RPT, RPT)])
        for m in range(RPT // 16):
            cnt_loc[pl.ds(m * 16, 16)] = (
                cnt_loc[pl.ds(m * 16, 16)]
                + cnt_loc[pl.ds(RPT + m * 16, 16)])
    for m in range(RPT // 16):
        cnt_loc[pl.ds(m * 16, 16)] = 1.0 / jnp.maximum(
            cnt_loc[pl.ds(m * 16, 16)], 1.0)

    def fin_slab(q, carry):
        rb = pl.multiple_of(rbase + q * FS, 8)
        pltpu.sync_copy(acc_sh.at[pl.ds(rb, FS)], obuf)
        for j in range(FS):
            ridx = jnp.full((16,), q * FS + j, jnp.int32)
            inv = plsc.load_gather(cnt_loc, [ridx])
            for k in range(D // 16):
                obuf[j, pl.ds(k * 16, 16)] = obuf[j, pl.ds(k * 16, 16)] * inv
        ob = pl.multiple_of(obase + q * FS, 8)
        pltpu.sync_copy(obuf, mean_all.at[pl.ds(ob, FS)])
        return carry

    lax.fori_loop(0, RPT // FS, fin_slab, 0)


def _dense_body(mean_ref, x_ref, wr_ref, br_ref, wt_ref, o_ref):
    dn = (((1,), (1,)), ((), ()))
    o_ref[...] = (
        lax.dot_general(mean_ref[...], wr_ref[...], dn,
                        preferred_element_type=jnp.float32)
        + br_ref[...]
        + lax.dot_general(x_ref[...], wt_ref[...], dn,
                          preferred_element_type=jnp.float32)
    )


def _dense(mean, x_dst, W_rel, b_rel, W_root):
    blk = 1000
    return pl.pallas_call(
        _dense_body,
        grid=(N_NODE // blk,),
        in_specs=[
            pl.BlockSpec((blk, D), lambda i: (i, 0)),
            pl.BlockSpec((blk, D), lambda i: (i, 0)),
            pl.BlockSpec((D, D), lambda i: (0, 0)),
            pl.BlockSpec((1, D), lambda i: (0, 0)),
            pl.BlockSpec((D, D), lambda i: (0, 0)),
        ],
        out_specs=pl.BlockSpec((blk, D), lambda i: (i, 0)),
        out_shape=jax.ShapeDtypeStruct((N_NODE, D), jnp.float32),
    )(mean, x_dst, W_rel, b_rel.reshape(1, D), W_root)


def kernel(x_user, x_item, edge_index_user_item, edge_index_item_user,
           W_rel_ui, b_rel_ui, W_root_ui, W_rel_iu, b_rel_iu, W_root_iu):
    pad = E_PAD - E
    pad_dst = jnp.full((pad,), PAD_N - 1, jnp.int32)

    def pad_type(edge_index, src_off):
        src = jnp.concatenate([edge_index[0].astype(jnp.int32) + src_off,
                               jnp.full((pad,), src_off, jnp.int32)])
        dst = jnp.concatenate([edge_index[1].astype(jnp.int32), pad_dst])
        return src, dst

    src_ui, dst_ui = pad_type(edge_index_user_item, 0)
    src_iu, dst_iu = pad_type(edge_index_item_user, N_NODE)
    src_all = jnp.concatenate([src_ui, src_iu])
    dst_all = jnp.concatenate([dst_ui, dst_iu])
    x_all = jnp.concatenate([x_user, x_item])

    zeros_f = jnp.zeros((FS, D), jnp.float32)
    zeros_n = jnp.zeros((PAD_N,), jnp.float32)

    mean_all, _ = _sc_aggregate(src_all, dst_all, x_all, zeros_f, zeros_n)
    mean_ui = mean_all[:N_NODE]
    mean_iu = mean_all[PAD_N:PAD_N + N_NODE]

    out_item = _dense(mean_ui, x_item, W_rel_ui, b_rel_ui, W_root_ui)
    out_user = _dense(mean_iu, x_user, W_rel_iu, b_rel_iu, W_root_iu)
    return (out_user, out_item)


# speedup vs baseline: 2.5439x; 1.0004x over previous
import functools

import jax
import jax.numpy as jnp
from jax import lax
from jax.experimental import pallas as pl
from jax.experimental.pallas import tpu as pltpu
from jax.experimental.pallas import tpu_sc as plsc

D = 128
N_NODE = 10000
PAD_N = 10240
NS = 16
RPT = PAD_N // NS
CH = 80
E = 320000
T_CH = 250
EPT = T_CH * CH
E_PAD = EPT * NS
FS = 80

_mesh = plsc.VectorSubcoreMesh(core_axis_name="c", subcore_axis_name="s")


@functools.partial(
    pl.kernel,
    out_type=[jax.ShapeDtypeStruct((2 * PAD_N, D), jnp.float32),
              jax.ShapeDtypeStruct((2, NS, PAD_N), jnp.float32)],
    mesh=_mesh,
    compiler_params=pltpu.CompilerParams(needs_layout_passes=False),
    scratch_types=[
        pltpu.VMEM((CH,), jnp.int32),
        pltpu.VMEM((CH,), jnp.int32),
        pltpu.VMEM((CH,), jnp.int32),
        pltpu.VMEM((CH,), jnp.int32),
        pltpu.VMEM((CH, D), jnp.float32),
        pltpu.VMEM((CH, D), jnp.float32),
        pltpu.VMEM((PAD_N,), jnp.float32),
        pltpu.VMEM_SHARED((PAD_N, D), jnp.float32),
        pltpu.SemaphoreType.DMA,
        pltpu.SemaphoreType.DMA,
    ],
)
def _sc_aggregate(src_all, dst_all, x_all, zeros_f, zeros_n,
                  mean_all, cnt_pub,
                  idx_sa, idx_da, idx_sb, idx_db, rows_a, rows_b,
                  cnt_loc, acc_sh, sem_a, sem_b):
    obuf = rows_a.at[pl.ds(0, FS)]
    c = lax.axis_index("c")
    s = lax.axis_index("s")
    rbase = pl.multiple_of(s * RPT, 8)
    obase = pl.multiple_of(c * PAD_N + s * RPT, 8)
    ebase = pl.multiple_of(c * E_PAD + s * EPT, 8)

    pltpu.sync_copy(zeros_f, obuf)

    def zero_slab(q, carry):
        rb = pl.multiple_of(rbase + q * FS, 8)
        pltpu.sync_copy(obuf, acc_sh.at[pl.ds(rb, FS)])
        return carry

    lax.fori_loop(0, RPT // FS, zero_slab, 0)
    pltpu.sync_copy(zeros_n, cnt_loc)
    plsc.subcore_barrier()

    ones = jnp.full((16,), 1.0, jnp.float32)

    def prefetch(i, idx_s, idx_d, rows, sem):
        base = pl.multiple_of(ebase + i * CH, 8)
        pltpu.sync_copy(src_all.at[pl.ds(base, CH)], idx_s)
        pltpu.async_copy(x_all.at[idx_s], rows, sem)
        pltpu.sync_copy(dst_all.at[pl.ds(base, CH)], idx_d)

    def finish(idx_d, rows, sem):
        for k in range(CH // 16):
            v = idx_d[pl.ds(k * 16, 16)]
            plsc.addupdate_scatter(cnt_loc, [v], ones)
        pltpu.make_async_copy(x_all.at[pl.ds(0, CH)], rows, sem).wait()
        pltpu.sync_copy(rows, acc_sh.at[idx_d], add=True)

    prefetch(0, idx_sa, idx_da, rows_a, sem_a)

    def pair(q, carry):
        prefetch(2 * q + 1, idx_sb, idx_db, rows_b, sem_b)
        finish(idx_da, rows_a, sem_a)
        nxt = jnp.minimum(2 * q + 2, T_CH - 1)
        prefetch(nxt, idx_sa, idx_da, rows_a, sem_a)
        finish(idx_db, rows_b, sem_b)
        return carry

    lax.fori_loop(0, T_CH // 2, pair, 0)
    pltpu.make_async_copy(x_all.at[pl.ds(0, CH)], rows_a, sem_a).wait()
    pltpu.sync_copy(cnt_loc, cnt_pub.at[c, s])
    plsc.subcore_barrier()

    pltpu.sync_copy(cnt_pub.at[c, 0, pl.ds(rbase, RPT)],
                    cnt_loc.at[pl.ds(0, RPT)])
    for t in range(1, NS):
        pltpu.sync_copy(cnt_pub.at[c, t, pl.ds(rbase, RPT)],
                        cnt_loc.at[pl.ds(RPT, RPT)])
        for m in range(RPT // 16):
            cnt_loc[pl.ds(m * 16, 16)] = (
                cnt_loc[pl.ds(m * 16, 16)]
                + cnt_loc[pl.ds(RPT + m * 16, 16)])
    for m in range(RPT // 16):
        cnt_loc[pl.ds(m * 16, 16)] = 1.0 / jnp.maximum(
            cnt_loc[pl.ds(m * 16, 16)], 1.0)

    def fin_slab(q, carry):
        rb = pl.multiple_of(rbase + q * FS, 8)
        pltpu.sync_copy(acc_sh.at[pl.ds(rb, FS)], obuf)
        for j in range(FS):
            ridx = jnp.full((16,), q * FS + j, jnp.int32)
            inv = plsc.load_gather(cnt_loc, [ridx])
            for k in range(D // 16):
                obuf[j, pl.ds(k * 16, 16)] = obuf[j, pl.ds(k * 16, 16)] * inv
        ob = pl.multiple_of(obase + q * FS, 8)
        pltpu.sync_copy(obuf, mean_all.at[pl.ds(ob, FS)])
        return carry

    lax.fori_loop(0, RPT // FS, fin_slab, 0)


def _dense_body(mean_ref, x_ref, wr_ref, br_ref, wt_ref, o_ref):
    dn = (((1,), (1,)), ((), ()))
    o_ref[...] = (
        lax.dot_general(mean_ref[...], wr_ref[...], dn,
                        preferred_element_type=jnp.float32)
        + br_ref[...]
        + lax.dot_general(x_ref[...], wt_ref[...], dn,
                          preferred_element_type=jnp.float32)
    )


def _dense(mean, x_dst, W_rel, b_rel, W_root):
    blk = 1000
    return pl.pallas_call(
        _dense_body,
        grid=(N_NODE // blk,),
        in_specs=[
            pl.BlockSpec((blk, D), lambda i: (i, 0)),
            pl.BlockSpec((blk, D), lambda i: (i, 0)),
            pl.BlockSpec((D, D), lambda i: (0, 0)),
            pl.BlockSpec((1, D), lambda i: (0, 0)),
            pl.BlockSpec((D, D), lambda i: (0, 0)),
        ],
        out_specs=pl.BlockSpec((blk, D), lambda i: (i, 0)),
        out_shape=jax.ShapeDtypeStruct((N_NODE, D), jnp.float32),
    )(mean, x_dst, W_rel, b_rel.reshape(1, D), W_root)


def kernel(x_user, x_item, edge_index_user_item, edge_index_item_user,
           W_rel_ui, b_rel_ui, W_root_ui, W_rel_iu, b_rel_iu, W_root_iu):
    pad = E_PAD - E
    pad_dst = jnp.full((pad,), PAD_N - 1, jnp.int32)

    def pad_type(edge_index, src_off):
        src = jnp.concatenate([edge_index[0].astype(jnp.int32) + src_off,
                               jnp.full((pad,), src_off, jnp.int32)])
        dst = jnp.concatenate([edge_index[1].astype(jnp.int32), pad_dst])
        return src, dst

    src_ui, dst_ui = pad_type(edge_index_user_item, 0)
    src_iu, dst_iu = pad_type(edge_index_item_user, N_NODE)
    src_all = jnp.concatenate([src_ui, src_iu])
    dst_all = jnp.concatenate([dst_ui, dst_iu])
    x_all = jnp.concatenate([x_user, x_item])

    zeros_f = jnp.zeros((FS, D), jnp.float32)
    zeros_n = jnp.zeros((PAD_N,), jnp.float32)

    mean_all, _ = _sc_aggregate(src_all, dst_all, x_all, zeros_f, zeros_n)
    mean_ui = mean_all[:N_NODE]
    mean_iu = mean_all[PAD_N:PAD_N + N_NODE]

    out_item = _dense(mean_ui, x_item, W_rel_ui, b_rel_ui, W_root_ui)
    out_user = _dense(mean_iu, x_user, W_rel_iu, b_rel_iu, W_root_iu)
    return (out_user, out_item)
